# Initial kernel scaffold; baseline (speedup 1.0000x reference)
import jax
import jax.numpy as jnp
from jax.experimental import pallas as pl


def kernel(x, time_mask_token, freq_mask_token):
    raise NotImplementedError("write your pallas kernel here")


# speedup vs baseline: 7.1565x; 7.1565x over previous
import functools

import numpy as np

import jax
import jax.numpy as jnp
from jax.experimental import pallas as pl
from jax.experimental.pallas import tpu as pltpu

_TIME_RATIO = 0.5
_FREQ_RATIO = 0.4


def _build_dft_matrix(n_vars: int, patch_len: int) -> np.ndarray:
    V, L = n_vars, patch_len
    nh = L // 2
    l = np.arange(L)
    cols = []
    sum_blk = np.zeros((V * L, V), np.float32)
    for v in range(V):
        sum_blk[v * L:(v + 1) * L, v] = 1.0
    cols.append(sum_blk)
    for k in range(1, nh + 1):
        blk = np.zeros((V * L, V), np.float32)
        for v in range(V):
            blk[v * L:(v + 1) * L, v] = np.cos(2.0 * np.pi * k * l / L)
        cols.append(blk)
    for k in range(1, nh):
        blk = np.zeros((V * L, V), np.float32)
        for v in range(V):
            blk[v * L:(v + 1) * L, v] = -np.sin(2.0 * np.pi * k * l / L)
        cols.append(blk)
    return np.concatenate(cols, axis=1)


def _build_expand_matrix(n_vars: int, patch_len: int) -> np.ndarray:
    V, L = n_vars, patch_len
    e = np.zeros((V, V * L), np.float32)
    for v in range(V):
        e[v, v * L:(v + 1) * L] = 1.0
    return e


def _score_body(n_vars, patch_len, x_ref, d_ref, cv_ref, fneg_ref):
    V, L = n_vars, patch_len
    nh = L // 2
    x = x_ref[...]
    g = jax.lax.dot_general(
        x, d_ref[...], (((1,), (0,)), ((), ())),
        precision=jax.lax.Precision.HIGHEST,
        preferred_element_type=jnp.float32)
    s1 = g[:, :V]
    sumsq_sp = s1 * s1
    fsum = jnp.abs(s1)
    for k in range(1, nh):
        re = g[:, (k) * V:(k + 1) * V]
        im = g[:, (nh + k) * V:(nh + k + 1) * V]
        p2 = re * re + im * im
        sumsq_sp = sumsq_sp + 2.0 * p2
        fsum = fsum + jnp.sqrt(p2)
    re_nyq = g[:, nh * V:(nh + 1) * V]
    sumsq_sp = sumsq_sp + re_nyq * re_nyq
    fsum = fsum + jnp.abs(re_nyq)
    s2 = sumsq_sp / L
    mean = s1 / L
    var = jnp.maximum(s2 - s1 * mean, 0.0) / (L - 1)
    cv_ref[...] = jnp.sqrt(var) / (mean + 1e-6)
    fneg_ref[...] = -fsum


def _sortable_u32(f):
    b = jax.lax.bitcast_convert_type(f, jnp.uint32)
    flip = jnp.where(b >= jnp.uint32(0x80000000),
                     jnp.uint32(0xFFFFFFFF), jnp.uint32(0x80000000))
    return b ^ flip


def _kth_largest_threshold(u, k):
    cand = jnp.zeros(u.shape[:1] + (1,) + u.shape[2:], jnp.uint32)
    for bit in range(31, -1, -1):
        trial = cand | jnp.uint32(1 << bit)
        cnt = jnp.sum((u >= trial).astype(jnp.int32), axis=1, keepdims=True)
        cand = jnp.where(cnt >= k, trial, cand)
    return cand


def _mask_body(k_t, k_f, cv_ref, fneg_ref, tt_ref, ft_ref, w_ref, c_ref):
    u_t = _sortable_u32(cv_ref[...])
    u_f = _sortable_u32(fneg_ref[...])
    sel_t = u_t >= _kth_largest_threshold(u_t, k_t)
    sel_f = u_f >= _kth_largest_threshold(u_f, k_f)
    tt = tt_ref[0, 0]
    ft = ft_ref[0, 0]
    keep_t = (~sel_t).astype(jnp.float32)
    keep_f = (~sel_f).astype(jnp.float32)
    w_ref[...] = 0.5 * (keep_t + keep_f)
    c_ref[...] = 0.5 * (tt * (1.0 - keep_t) + ft * (1.0 - keep_f))


def _apply_body(x_ref, w_ref, c_ref, e_ref, o_ref):
    e = e_ref[...]
    dn = (((1,), (0,)), ((), ()))
    wx = jax.lax.dot_general(w_ref[...], e, dn,
                             precision=jax.lax.Precision.HIGHEST,
                             preferred_element_type=jnp.float32)
    cx = jax.lax.dot_general(c_ref[...], e, dn,
                             precision=jax.lax.Precision.HIGHEST,
                             preferred_element_type=jnp.float32)
    o_ref[...] = x_ref[...] * wx + cx


def kernel(x, time_mask_token, freq_mask_token):
    bs, P, V, L = x.shape
    k_t = int(P * _TIME_RATIO)
    k_f = int(P * _FREQ_RATIO)
    rows = bs * P
    x2 = x.reshape(rows, V * L)
    d_mat = jnp.asarray(_build_dft_matrix(V, L))
    e_mat = jnp.asarray(_build_expand_matrix(V, L))
    dcols = d_mat.shape[1]

    R = 2048
    nblk = rows // R

    cv, fneg = pl.pallas_call(
        functools.partial(_score_body, V, L),
        grid=(nblk,),
        in_specs=[
            pl.BlockSpec((R, V * L), lambda i: (i, 0)),
            pl.BlockSpec((V * L, dcols), lambda i: (0, 0)),
        ],
        out_specs=[
            pl.BlockSpec((R, V), lambda i: (i, 0)),
            pl.BlockSpec((R, V), lambda i: (i, 0)),
        ],
        out_shape=[
            jax.ShapeDtypeStruct((rows, V), jnp.float32),
            jax.ShapeDtypeStruct((rows, V), jnp.float32),
        ],
    )(x2, d_mat)

    BB = 8
    w3, c3 = pl.pallas_call(
        functools.partial(_mask_body, k_t, k_f),
        grid=(bs // BB,),
        in_specs=[
            pl.BlockSpec((BB, P, V), lambda i: (i, 0, 0)),
            pl.BlockSpec((BB, P, V), lambda i: (i, 0, 0)),
            pl.BlockSpec((1, 1), lambda i: (0, 0)),
            pl.BlockSpec((1, 1), lambda i: (0, 0)),
        ],
        out_specs=[
            pl.BlockSpec((BB, P, V), lambda i: (i, 0, 0)),
            pl.BlockSpec((BB, P, V), lambda i: (i, 0, 0)),
        ],
        out_shape=[
            jax.ShapeDtypeStruct((bs, P, V), jnp.float32),
            jax.ShapeDtypeStruct((bs, P, V), jnp.float32),
        ],
    )(cv.reshape(bs, P, V), fneg.reshape(bs, P, V),
      time_mask_token.reshape(1, 1), freq_mask_token.reshape(1, 1))

    out2 = pl.pallas_call(
        _apply_body,
        grid=(nblk,),
        in_specs=[
            pl.BlockSpec((R, V * L), lambda i: (i, 0)),
            pl.BlockSpec((R, V), lambda i: (i, 0)),
            pl.BlockSpec((R, V), lambda i: (i, 0)),
            pl.BlockSpec((V, V * L), lambda i: (0, 0)),
        ],
        out_specs=pl.BlockSpec((R, V * L), lambda i: (i, 0)),
        out_shape=jax.ShapeDtypeStruct((rows, V * L), jnp.float32),
    )(x2, w3.reshape(rows, V), c3.reshape(rows, V), e_mat)

    return out2.reshape(bs, P, V, L)


# speedup vs baseline: 11.3572x; 1.5870x over previous
import functools

import numpy as np

import jax
import jax.numpy as jnp
from jax.experimental import pallas as pl
from jax.experimental.pallas import tpu as pltpu

_TIME_RATIO = 0.5
_FREQ_RATIO = 0.4
_SEARCH_BITS = 26


def _build_dft_matrix(n_vars: int, patch_len: int) -> np.ndarray:
    V, L = n_vars, patch_len
    nh = L // 2
    l = np.arange(L)
    cols = []
    sum_blk = np.zeros((V * L, V), np.float32)
    for v in range(V):
        sum_blk[v * L:(v + 1) * L, v] = 1.0
    cols.append(sum_blk)
    for k in range(1, nh + 1):
        blk = np.zeros((V * L, V), np.float32)
        for v in range(V):
            blk[v * L:(v + 1) * L, v] = np.cos(2.0 * np.pi * k * l / L)
        cols.append(blk)
    for k in range(1, nh):
        blk = np.zeros((V * L, V), np.float32)
        for v in range(V):
            blk[v * L:(v + 1) * L, v] = -np.sin(2.0 * np.pi * k * l / L)
        cols.append(blk)
    return np.concatenate(cols, axis=1)


def _build_expand_matrix(n_vars: int, patch_len: int) -> np.ndarray:
    V, L = n_vars, patch_len
    e = np.zeros((V, V * L), np.float32)
    for v in range(V):
        e[v, v * L:(v + 1) * L] = 1.0
    return e


def _score_body(n_vars, patch_len, n_patch, b_per_blk,
                x_ref, d_ref, cv_ref, fneg_ref, cvt_ref, fnegt_ref):
    V, L, P = n_vars, patch_len, n_patch
    nh = L // 2
    x = x_ref[...]
    d = d_ref[...]
    x_hi = x.astype(jnp.bfloat16)
    x_lo = (x - x_hi.astype(jnp.float32)).astype(jnp.bfloat16)
    d_hi = d.astype(jnp.bfloat16)
    d_lo = (d - d_hi.astype(jnp.float32)).astype(jnp.bfloat16)
    dn = (((1,), (0,)), ((), ()))
    g = (jax.lax.dot_general(x_hi, d_hi, dn, preferred_element_type=jnp.float32)
         + (jax.lax.dot_general(x_hi, d_lo, dn, preferred_element_type=jnp.float32)
            + jax.lax.dot_general(x_lo, d_hi, dn, preferred_element_type=jnp.float32)))
    s1 = g[:, :V]
    sumsq_sp = s1 * s1
    fsum = jnp.abs(s1)
    for k in range(1, nh):
        re = g[:, (k) * V:(k + 1) * V]
        im = g[:, (nh + k) * V:(nh + k + 1) * V]
        p2 = re * re + im * im
        sumsq_sp = sumsq_sp + 2.0 * p2
        fsum = fsum + jnp.sqrt(p2)
    re_nyq = g[:, nh * V:(nh + 1) * V]
    sumsq_sp = sumsq_sp + re_nyq * re_nyq
    fsum = fsum + jnp.abs(re_nyq)
    s2 = sumsq_sp / L
    mean = s1 / L
    var = jnp.maximum(s2 - s1 * mean, 0.0) / (L - 1)
    cv = jnp.sqrt(var) / (mean + 1e-6)
    fneg = -fsum
    cv_ref[...] = cv
    fneg_ref[...] = fneg
    for j in range(b_per_blk):
        cvt_ref[j] = jnp.transpose(cv[j * P:(j + 1) * P, :])
        fnegt_ref[j] = jnp.transpose(fneg[j * P:(j + 1) * P, :])


def _sortable_u32(f):
    b = jax.lax.bitcast_convert_type(f, jnp.uint32)
    flip = jnp.where(b >= jnp.uint32(0x80000000),
                     jnp.uint32(0xFFFFFFFF), jnp.uint32(0x80000000))
    return b ^ flip


def _kth_threshold_lanes(u, k):
    cand = jnp.zeros(u.shape[:2] + (1,), jnp.uint32)
    for bit in range(31, 31 - _SEARCH_BITS, -1):
        trial = cand | jnp.uint32(1 << bit)
        cnt = jnp.sum((u >= trial).astype(jnp.int32), axis=2, keepdims=True)
        cand = jnp.where(cnt >= k, trial, cand)
    return cand


def _thresh_body(k_t, k_f, cvt_ref, fnegt_ref, tt_ref, tf_ref):
    u_t = _sortable_u32(cvt_ref[...])
    u_f = _sortable_u32(fnegt_ref[...])
    tt_ref[...] = jnp.transpose(_kth_threshold_lanes(u_t, k_t), (0, 2, 1))
    tf_ref[...] = jnp.transpose(_kth_threshold_lanes(u_f, k_f), (0, 2, 1))


def _apply_body(b_per_blk, n_patch, x_ref, cv_ref, fneg_ref, thrt_ref,
                thrf_ref, e_ref, tt_ref, ft_ref, o_ref):
    B, P = b_per_blk, n_patch
    V = cv_ref.shape[1]
    u_t = _sortable_u32(cv_ref[...]).reshape(B, P, V)
    u_f = _sortable_u32(fneg_ref[...]).reshape(B, P, V)
    sel_t = u_t >= thrt_ref[...]
    sel_f = u_f >= thrf_ref[...]
    kt = jnp.where(sel_t, 0.0, 1.0).reshape(B * P, V)
    kf = jnp.where(sel_f, 0.0, 1.0).reshape(B * P, V)
    e = e_ref[...]
    dn = (((1,), (0,)), ((), ()))
    a = jax.lax.dot_general(kt, e, dn, preferred_element_type=jnp.float32)
    b = jax.lax.dot_general(kf, e, dn, preferred_element_type=jnp.float32)
    tt = tt_ref[0, 0]
    ft = ft_ref[0, 0]
    o_ref[...] = 0.5 * (x_ref[...] * (a + b) + tt * (1.0 - a) + ft * (1.0 - b))


def kernel(x, time_mask_token, freq_mask_token):
    bs, P, V, L = x.shape
    k_t = int(P * _TIME_RATIO)
    k_f = int(P * _FREQ_RATIO)
    rows = bs * P
    x2 = x.reshape(rows, V * L)
    d_mat = jnp.asarray(_build_dft_matrix(V, L))
    e_mat = jnp.asarray(_build_expand_matrix(V, L))
    dcols = d_mat.shape[1]

    BPB = 4
    R = BPB * P
    nblk = rows // R

    cv, fneg, cvt, fnegt = pl.pallas_call(
        functools.partial(_score_body, V, L, P, BPB),
        grid=(nblk,),
        in_specs=[
            pl.BlockSpec((R, V * L), lambda i: (i, 0)),
            pl.BlockSpec((V * L, dcols), lambda i: (0, 0)),
        ],
        out_specs=[
            pl.BlockSpec((R, V), lambda i: (i, 0)),
            pl.BlockSpec((R, V), lambda i: (i, 0)),
            pl.BlockSpec((BPB, V, P), lambda i: (i, 0, 0)),
            pl.BlockSpec((BPB, V, P), lambda i: (i, 0, 0)),
        ],
        out_shape=[
            jax.ShapeDtypeStruct((rows, V), jnp.float32),
            jax.ShapeDtypeStruct((rows, V), jnp.float32),
            jax.ShapeDtypeStruct((bs, V, P), jnp.float32),
            jax.ShapeDtypeStruct((bs, V, P), jnp.float32),
        ],
    )(x2, d_mat)

    thrt, thrf = pl.pallas_call(
        functools.partial(_thresh_body, k_t, k_f),
        in_specs=[
            pl.BlockSpec((bs, V, P), lambda: (0, 0, 0)),
            pl.BlockSpec((bs, V, P), lambda: (0, 0, 0)),
        ],
        out_specs=[
            pl.BlockSpec((bs, 1, V), lambda: (0, 0, 0)),
            pl.BlockSpec((bs, 1, V), lambda: (0, 0, 0)),
        ],
        out_shape=[
            jax.ShapeDtypeStruct((bs, 1, V), jnp.uint32),
            jax.ShapeDtypeStruct((bs, 1, V), jnp.uint32),
        ],
    )(cvt, fnegt)

    out2 = pl.pallas_call(
        functools.partial(_apply_body, BPB, P),
        grid=(nblk,),
        in_specs=[
            pl.BlockSpec((R, V * L), lambda i: (i, 0)),
            pl.BlockSpec((R, V), lambda i: (i, 0)),
            pl.BlockSpec((R, V), lambda i: (i, 0)),
            pl.BlockSpec((BPB, 1, V), lambda i: (i, 0, 0)),
            pl.BlockSpec((BPB, 1, V), lambda i: (i, 0, 0)),
            pl.BlockSpec((V, V * L), lambda i: (0, 0)),
            pl.BlockSpec((1, 1), lambda i: (0, 0)),
            pl.BlockSpec((1, 1), lambda i: (0, 0)),
        ],
        out_specs=pl.BlockSpec((R, V * L), lambda i: (i, 0)),
        out_shape=jax.ShapeDtypeStruct((rows, V * L), jnp.float32),
    )(x2, cv, fneg, thrt, thrf, e_mat,
      time_mask_token.reshape(1, 1), freq_mask_token.reshape(1, 1))

    return out2.reshape(bs, P, V, L)


# speedup vs baseline: 11.4583x; 1.0089x over previous
import functools

import numpy as np

import jax
import jax.numpy as jnp
from jax.experimental import pallas as pl
from jax.experimental.pallas import tpu as pltpu

_TIME_RATIO = 0.5
_FREQ_RATIO = 0.4
_SEARCH_BITS = 26


def _build_dft_matrix(n_vars: int, patch_len: int) -> np.ndarray:
    V, L = n_vars, patch_len
    nh = L // 2
    l = np.arange(L)
    cols = []
    sum_blk = np.zeros((V * L, V), np.float32)
    for v in range(V):
        sum_blk[v * L:(v + 1) * L, v] = 1.0
    cols.append(sum_blk)
    for k in range(1, nh + 1):
        blk = np.zeros((V * L, V), np.float32)
        for v in range(V):
            blk[v * L:(v + 1) * L, v] = np.cos(2.0 * np.pi * k * l / L)
        cols.append(blk)
    for k in range(1, nh):
        blk = np.zeros((V * L, V), np.float32)
        for v in range(V):
            blk[v * L:(v + 1) * L, v] = -np.sin(2.0 * np.pi * k * l / L)
        cols.append(blk)
    return np.concatenate(cols, axis=1)


def _build_expand_matrix(n_vars: int, patch_len: int) -> np.ndarray:
    V, L = n_vars, patch_len
    e = np.zeros((V, V * L), np.float32)
    for v in range(V):
        e[v, v * L:(v + 1) * L] = 1.0
    return e


def _score_body(n_vars, patch_len, n_patch, b_per_blk,
                x_ref, d_ref, cvt_ref, fnegt_ref):
    V, L, P = n_vars, patch_len, n_patch
    nh = L // 2
    x = x_ref[...]
    d = d_ref[...]
    x_hi = x.astype(jnp.bfloat16)
    x_lo = (x - x_hi.astype(jnp.float32)).astype(jnp.bfloat16)
    d_hi = d.astype(jnp.bfloat16)
    d_lo = (d - d_hi.astype(jnp.float32)).astype(jnp.bfloat16)
    dn = (((1,), (0,)), ((), ()))
    g = (jax.lax.dot_general(x_hi, d_hi, dn, preferred_element_type=jnp.float32)
         + (jax.lax.dot_general(x_hi, d_lo, dn, preferred_element_type=jnp.float32)
            + jax.lax.dot_general(x_lo, d_hi, dn, preferred_element_type=jnp.float32)))
    s1 = g[:, :V]
    sumsq_sp = s1 * s1
    fsum = jnp.abs(s1)
    for k in range(1, nh):
        re = g[:, (k) * V:(k + 1) * V]
        im = g[:, (nh + k) * V:(nh + k + 1) * V]
        p2 = re * re + im * im
        sumsq_sp = sumsq_sp + 2.0 * p2
        fsum = fsum + jnp.sqrt(p2)
    re_nyq = g[:, nh * V:(nh + 1) * V]
    sumsq_sp = sumsq_sp + re_nyq * re_nyq
    fsum = fsum + jnp.abs(re_nyq)
    s2 = sumsq_sp / L
    mean = s1 / L
    var = jnp.maximum(s2 - s1 * mean, 0.0) / (L - 1)
    cv = jnp.sqrt(var) / (mean + 1e-6)
    fneg = -fsum
    for j in range(b_per_blk):
        cvt_ref[j] = jnp.transpose(cv[j * P:(j + 1) * P, :])
        fnegt_ref[j] = jnp.transpose(fneg[j * P:(j + 1) * P, :])


def _sortable_u32(f):
    b = jax.lax.bitcast_convert_type(f, jnp.uint32)
    flip = jnp.where(b >= jnp.uint32(0x80000000),
                     jnp.uint32(0xFFFFFFFF), jnp.uint32(0x80000000))
    return b ^ flip


def _kth_threshold_lanes(u, k):
    cand = jnp.zeros(u.shape[:2] + (1,), jnp.uint32)
    for bit in range(31, 31 - _SEARCH_BITS, -1):
        trial = cand | jnp.uint32(1 << bit)
        cnt = jnp.sum((u >= trial).astype(jnp.int32), axis=2, keepdims=True)
        cand = jnp.where(cnt >= k, trial, cand)
    return cand


def _thresh_body(k_t, k_f, cvt_ref, fnegt_ref, tt_ref, tf_ref):
    u_t = _sortable_u32(cvt_ref[...])
    u_f = _sortable_u32(fnegt_ref[...])
    tt_ref[...] = _kth_threshold_lanes(u_t, k_t)
    tf_ref[...] = _kth_threshold_lanes(u_f, k_f)


def _apply_body(b_per_blk, n_patch, x_ref, cvt_ref, fnegt_ref, thrt_ref,
                thrf_ref, e_ref, tt_ref, ft_ref, o_ref):
    B, P = b_per_blk, n_patch
    u_t = _sortable_u32(cvt_ref[...])
    u_f = _sortable_u32(fnegt_ref[...])
    kt_t = jnp.where(u_t >= thrt_ref[...], 0.0, 1.0)
    kf_t = jnp.where(u_f >= thrf_ref[...], 0.0, 1.0)
    e = e_ref[...]
    dn0 = (((0,), (0,)), ((), ()))
    tt = tt_ref[0, 0]
    ft = ft_ref[0, 0]
    for j in range(B):
        a = jax.lax.dot_general(kt_t[j], e, dn0,
                                preferred_element_type=jnp.float32)
        b = jax.lax.dot_general(kf_t[j], e, dn0,
                                preferred_element_type=jnp.float32)
        sl = pl.ds(j * P, P)
        o_ref[sl, :] = 0.5 * (x_ref[sl, :] * (a + b)
                              + tt * (1.0 - a) + ft * (1.0 - b))


def kernel(x, time_mask_token, freq_mask_token):
    bs, P, V, L = x.shape
    k_t = int(P * _TIME_RATIO)
    k_f = int(P * _FREQ_RATIO)
    rows = bs * P
    x2 = x.reshape(rows, V * L)
    d_mat = jnp.asarray(_build_dft_matrix(V, L))
    e_mat = jnp.asarray(_build_expand_matrix(V, L))
    dcols = d_mat.shape[1]

    BPB = 4
    R = BPB * P
    nblk = rows // R

    cvt, fnegt = pl.pallas_call(
        functools.partial(_score_body, V, L, P, BPB),
        grid=(nblk,),
        in_specs=[
            pl.BlockSpec((R, V * L), lambda i: (i, 0)),
            pl.BlockSpec((V * L, dcols), lambda i: (0, 0)),
        ],
        out_specs=[
            pl.BlockSpec((BPB, V, P), lambda i: (i, 0, 0)),
            pl.BlockSpec((BPB, V, P), lambda i: (i, 0, 0)),
        ],
        out_shape=[
            jax.ShapeDtypeStruct((bs, V, P), jnp.float32),
            jax.ShapeDtypeStruct((bs, V, P), jnp.float32),
        ],
    )(x2, d_mat)

    thrt, thrf = pl.pallas_call(
        functools.partial(_thresh_body, k_t, k_f),
        in_specs=[
            pl.BlockSpec((bs, V, P), lambda: (0, 0, 0)),
            pl.BlockSpec((bs, V, P), lambda: (0, 0, 0)),
        ],
        out_specs=[
            pl.BlockSpec((bs, V, 1), lambda: (0, 0, 0)),
            pl.BlockSpec((bs, V, 1), lambda: (0, 0, 0)),
        ],
        out_shape=[
            jax.ShapeDtypeStruct((bs, V, 1), jnp.uint32),
            jax.ShapeDtypeStruct((bs, V, 1), jnp.uint32),
        ],
    )(cvt, fnegt)

    out2 = pl.pallas_call(
        functools.partial(_apply_body, BPB, P),
        grid=(nblk,),
        in_specs=[
            pl.BlockSpec((R, V * L), lambda i: (i, 0)),
            pl.BlockSpec((BPB, V, P), lambda i: (i, 0, 0)),
            pl.BlockSpec((BPB, V, P), lambda i: (i, 0, 0)),
            pl.BlockSpec((BPB, V, 1), lambda i: (i, 0, 0)),
            pl.BlockSpec((BPB, V, 1), lambda i: (i, 0, 0)),
            pl.BlockSpec((V, V * L), lambda i: (0, 0)),
            pl.BlockSpec((1, 1), lambda i: (0, 0)),
            pl.BlockSpec((1, 1), lambda i: (0, 0)),
        ],
        out_specs=pl.BlockSpec((R, V * L), lambda i: (i, 0)),
        out_shape=jax.ShapeDtypeStruct((rows, V * L), jnp.float32),
    )(x2, cvt, fnegt, thrt, thrf, e_mat,
      time_mask_token.reshape(1, 1), freq_mask_token.reshape(1, 1))

    return out2.reshape(bs, P, V, L)
